# Initial kernel scaffold; baseline (speedup 1.0000x reference)
import jax
import jax.numpy as jnp
from jax.experimental import pallas as pl


def kernel(val, index):
    raise NotImplementedError("write your pallas kernel here")


# speedup vs baseline: 4.8627x; 4.8627x over previous
import functools

import jax
import jax.numpy as jnp
from jax import lax
from jax.experimental import pallas as pl
from jax.experimental.pallas import tpu as pltpu
from jax.experimental.pallas import tpu_sc as plsc

N = 6291456
TOTAL = 25165824
W = 1835008
NWIN = 14
LASTW = TOTAL - 13 * W
NPASS = 7
NT = 16
PER_TILE = N // NT
CHUNK = 2048
NCHUNK = PER_TILE // CHUNK
ZB = 2048
SLICE = W // NT
DUMPMASK = (1 << 20) - 1


def _body(idx_hbm, val_hbm, out_hbm, win):
    pl.run_scoped(
        functools.partial(_scan, idx_hbm, val_hbm, out_hbm, win),
        pltpu.VMEM((CHUNK,), jnp.int32),
        pltpu.VMEM((CHUNK,), jnp.float32),
        pltpu.VMEM((CHUNK,), jnp.int32),
        pltpu.VMEM((CHUNK,), jnp.float32),
        pltpu.VMEM((ZB,), jnp.float32),
    )


def _scan(idx_hbm, val_hbm, out_hbm, win, idx_st, val_st, idxw_st, valw_st,
          zbuf):
    c = lax.axis_index("c")
    s = lax.axis_index("s")
    z16 = jnp.zeros((16,), jnp.float32)

    def zb_init(i, carry):
        zbuf[pl.ds(i * 16, 16)] = z16
        return carry

    lax.fori_loop(0, ZB // 16, zb_init, 0)

    def pass_body(p, carry):
        wid = 2 * p + c
        base = wid * W
        bound = jnp.where(wid == NWIN - 1, jnp.uint32(LASTW), jnp.uint32(W))

        for j in range(SLICE // ZB):
            pltpu.sync_copy(zbuf, win.at[pl.ds(s * SLICE + j * ZB, ZB)])
        plsc.subcore_barrier()

        def chunk_body(k, carry1):
            off = s * PER_TILE + k * CHUNK
            pltpu.sync_copy(idx_hbm.at[pl.ds(off, CHUNK)], idx_st)
            pltpu.sync_copy(val_hbm.at[pl.ds(off, CHUNK)], val_st)

            def vec_body(i, carry2):
                iv = idx_st[pl.ds(i * 16, 16)]
                vv = val_st[pl.ds(i * 16, 16)]
                rel = iv - base
                m = plsc.bitcast(rel, jnp.uint32) < bound
                idxw_st[pl.ds(i * 16, 16)] = jnp.where(m, rel, rel & DUMPMASK)
                valw_st[pl.ds(i * 16, 16)] = jnp.where(m, vv, 0.0)
                return carry2

            lax.fori_loop(0, CHUNK // 16, vec_body, 0)
            pltpu.sync_copy(valw_st, win.at[idxw_st], add=True)
            return carry1

        lax.fori_loop(0, NCHUNK, chunk_body, 0)
        plsc.subcore_barrier()

        @pl.when(wid != NWIN - 1)
        def _():
            pltpu.sync_copy(win.at[pl.ds(s * SLICE, SLICE)],
                            out_hbm.at[pl.ds(base + s * SLICE, SLICE)])

        @pl.when(wid == NWIN - 1)
        def _():
            lsl = LASTW // NT
            pltpu.sync_copy(win.at[pl.ds(s * lsl, lsl)],
                            out_hbm.at[pl.ds(base + s * lsl, lsl)])

        return carry

    lax.fori_loop(0, NPASS, pass_body, 0)


@jax.jit
def _unpool(idx, val):
    mesh = plsc.VectorSubcoreMesh(core_axis_name="c", subcore_axis_name="s")
    call = pl.kernel(
        _body,
        out_type=jax.ShapeDtypeStruct((TOTAL,), jnp.float32),
        mesh=mesh,
        scratch_types=[
            pltpu.VMEM_SHARED((W,), jnp.float32),
        ],
    )
    return call(idx, val)


def kernel(val, index):
    idx = jnp.reshape(index, (-1,)).astype(jnp.int32)
    v = jnp.reshape(val, (-1,))
    flat = _unpool(idx, v)
    return jnp.reshape(flat, (1, 512, 512, 96))


# speedup vs baseline: 10.3680x; 2.1321x over previous
import functools

import jax
import jax.numpy as jnp
from jax import lax
from jax.experimental import pallas as pl
from jax.experimental.pallas import tpu as pltpu
from jax.experimental.pallas import tpu_sc as plsc

N = 6291456
TOTAL = 25165824
W = 1835008
NWIN = 14
LASTW = TOTAL - 13 * W
NPASS = 7
NT = 16
PER_TILE = N // NT
CHUNK = 2048
NCHUNK = PER_TILE // CHUNK
NPAIR = NCHUNK // 2
ZB = 2048
SLICE = W // NT
DUMPMASK = (1 << 20) - 1


def _body(idx_hbm, val_hbm, out_hbm, win):
    pl.run_scoped(
        functools.partial(_scan, idx_hbm, val_hbm, out_hbm, win),
        pltpu.VMEM((CHUNK,), jnp.int32),
        pltpu.VMEM((CHUNK,), jnp.float32),
        pltpu.VMEM((CHUNK,), jnp.int32),
        pltpu.VMEM((CHUNK,), jnp.float32),
        pltpu.VMEM((ZB,), jnp.float32),
        pltpu.SemaphoreType.DMA,
        pltpu.SemaphoreType.DMA,
        pltpu.SemaphoreType.DMA,
        pltpu.SemaphoreType.DMA,
    )


def _scan(idx_hbm, val_hbm, out_hbm, win, idx_a, val_a, idx_b, val_b,
          zbuf, lsem_a, lsem_b, ssem_a, ssem_b):
    c = lax.axis_index("c")
    s = lax.axis_index("s")
    z16 = jnp.zeros((16,), jnp.float32)

    def zb_init(i, carry):
        zbuf[pl.ds(i * 16, 16)] = z16
        return carry

    lax.fori_loop(0, ZB // 16, zb_init, 0)

    def load(k, ibuf, vbuf, sem):
        off = s * PER_TILE + k * CHUNK
        pltpu.async_copy(idx_hbm.at[pl.ds(off, CHUNK)], ibuf, sem)
        pltpu.async_copy(val_hbm.at[pl.ds(off, CHUNK)], vbuf, sem)

    def wait_load(k, ibuf, vbuf, sem):
        del k
        pltpu.make_async_copy(idx_hbm.at[pl.ds(0, CHUNK)], ibuf, sem).wait()
        pltpu.make_async_copy(val_hbm.at[pl.ds(0, CHUNK)], vbuf, sem).wait()

    def pass_body(p, carry):
        wid = 2 * p + c
        base = wid * W
        bound = jnp.where(wid == NWIN - 1, jnp.uint32(LASTW), jnp.uint32(W))

        for j in range(SLICE // ZB):
            pltpu.sync_copy(zbuf, win.at[pl.ds(s * SLICE + j * ZB, ZB)])
        plsc.subcore_barrier()

        load(0, idx_a, val_a, lsem_a)
        load(1, idx_b, val_b, lsem_b)

        def compute(ibuf, vbuf):
            def vec_body(i, carry2):
                for u in range(4):
                    o = (i * 4 + u) * 16
                    iv = ibuf[pl.ds(o, 16)]
                    vv = vbuf[pl.ds(o, 16)]
                    rel = iv - base
                    m = plsc.bitcast(rel, jnp.uint32) < bound
                    ibuf[pl.ds(o, 16)] = jnp.where(m, rel, rel & DUMPMASK)
                    vbuf[pl.ds(o, 16)] = jnp.where(m, vv, 0.0)
                return carry2

            lax.fori_loop(0, CHUNK // 64, vec_body, 0)

        def wait_scatter(vbuf, sem):
            pltpu.make_async_copy(vbuf, win.at[pl.ds(0, CHUNK)], sem).wait()

        def pair_body(q, carry1):
            wait_load(2 * q, idx_a, val_a, lsem_a)
            compute(idx_a, val_a)
            pltpu.async_copy(val_a, win.at[idx_a], ssem_a, add=True)

            wait_load(2 * q + 1, idx_b, val_b, lsem_b)
            compute(idx_b, val_b)
            pltpu.async_copy(val_b, win.at[idx_b], ssem_b, add=True)

            @pl.when(q + 1 < NPAIR)
            def _():
                wait_scatter(val_a, ssem_a)
                load(2 * q + 2, idx_a, val_a, lsem_a)
                wait_scatter(val_b, ssem_b)
                load(2 * q + 3, idx_b, val_b, lsem_b)

            return carry1

        lax.fori_loop(0, NPAIR, pair_body, 0)
        wait_scatter(val_a, ssem_a)
        wait_scatter(val_b, ssem_b)
        plsc.subcore_barrier()

        @pl.when(wid != NWIN - 1)
        def _():
            pltpu.sync_copy(win.at[pl.ds(s * SLICE, SLICE)],
                            out_hbm.at[pl.ds(base + s * SLICE, SLICE)])

        @pl.when(wid == NWIN - 1)
        def _():
            lsl = LASTW // NT
            pltpu.sync_copy(win.at[pl.ds(s * lsl, lsl)],
                            out_hbm.at[pl.ds(base + s * lsl, lsl)])

        return carry

    lax.fori_loop(0, NPASS, pass_body, 0)


@jax.jit
def _unpool(idx, val):
    mesh = plsc.VectorSubcoreMesh(core_axis_name="c", subcore_axis_name="s")
    call = pl.kernel(
        _body,
        out_type=jax.ShapeDtypeStruct((TOTAL,), jnp.float32),
        mesh=mesh,
        scratch_types=[
            pltpu.VMEM_SHARED((W,), jnp.float32),
        ],
    )
    return call(idx, val)


def kernel(val, index):
    idx = jnp.reshape(index, (-1,)).astype(jnp.int32)
    v = jnp.reshape(val, (-1,))
    flat = _unpool(idx, v)
    return jnp.reshape(flat, (1, 512, 512, 96))


# speedup vs baseline: 11.2407x; 1.0842x over previous
import functools

import jax
import jax.numpy as jnp
from jax import lax
from jax.experimental import pallas as pl
from jax.experimental.pallas import tpu as pltpu
from jax.experimental.pallas import tpu_sc as plsc

N = 6291456
TOTAL = 25165824
W = 1835008
NWIN = 14
LASTW = TOTAL - 13 * W
NPASS = 7
NT = 16
PER_TILE = N // NT
CHUNK = 3072
NCHUNK = PER_TILE // CHUNK
NPAIR = NCHUNK // 2
ZB = 2048
SLICE = W // NT
DUMPMASK = (1 << 20) - 1


def _body(idx_hbm, val_hbm, out_hbm, win):
    pl.run_scoped(
        functools.partial(_scan, idx_hbm, val_hbm, out_hbm, win),
        pltpu.VMEM((CHUNK,), jnp.int32),
        pltpu.VMEM((CHUNK,), jnp.float32),
        pltpu.VMEM((CHUNK,), jnp.int32),
        pltpu.VMEM((CHUNK,), jnp.float32),
        pltpu.VMEM((ZB,), jnp.float32),
        pltpu.SemaphoreType.DMA,
        pltpu.SemaphoreType.DMA,
        pltpu.SemaphoreType.DMA,
        pltpu.SemaphoreType.DMA,
        pltpu.SemaphoreType.DMA,
    )


def _scan(idx_hbm, val_hbm, out_hbm, win, idx_a, val_a, idx_b, val_b,
          zbuf, lsem_a, lsem_b, ssem_a, ssem_b, zsem):
    c = lax.axis_index("c")
    s = lax.axis_index("s")
    z16 = jnp.zeros((16,), jnp.float32)

    def zb_init(i, carry):
        zbuf[pl.ds(i * 16, 16)] = z16
        return carry

    lax.fori_loop(0, ZB // 16, zb_init, 0)

    def load(k, ibuf, vbuf, sem):
        off = s * PER_TILE + k * CHUNK
        pltpu.async_copy(idx_hbm.at[pl.ds(off, CHUNK)], ibuf, sem)
        pltpu.async_copy(val_hbm.at[pl.ds(off, CHUNK)], vbuf, sem)

    def wait_load(k, ibuf, vbuf, sem):
        del k
        pltpu.make_async_copy(idx_hbm.at[pl.ds(0, CHUNK)], ibuf, sem).wait()
        pltpu.make_async_copy(val_hbm.at[pl.ds(0, CHUNK)], vbuf, sem).wait()

    def pass_body(p, carry):
        wid = 2 * p + c
        base = wid * W
        bound = jnp.where(wid == NWIN - 1, jnp.uint32(LASTW), jnp.uint32(W))

        for j in range(SLICE // ZB):
            pltpu.async_copy(zbuf, win.at[pl.ds(s * SLICE + j * ZB, ZB)],
                             zsem)
        for j in range(SLICE // ZB):
            pltpu.make_async_copy(
                zbuf, win.at[pl.ds(s * SLICE + j * ZB, ZB)], zsem).wait()
        plsc.subcore_barrier()

        load(0, idx_a, val_a, lsem_a)
        load(1, idx_b, val_b, lsem_b)

        def compute(ibuf, vbuf):
            def vec_body(i, carry2):
                for u in range(4):
                    o = (i * 4 + u) * 16
                    iv = ibuf[pl.ds(o, 16)]
                    vv = vbuf[pl.ds(o, 16)]
                    rel = iv - base
                    m = plsc.bitcast(rel, jnp.uint32) < bound
                    ibuf[pl.ds(o, 16)] = jnp.where(m, rel, rel & DUMPMASK)
                    vbuf[pl.ds(o, 16)] = jnp.where(m, vv, 0.0)
                return carry2

            lax.fori_loop(0, CHUNK // 64, vec_body, 0)

        def wait_scatter(vbuf, sem):
            pltpu.make_async_copy(vbuf, win.at[pl.ds(0, CHUNK)], sem).wait()

        def pair_body(q, carry1):
            wait_load(2 * q, idx_a, val_a, lsem_a)
            compute(idx_a, val_a)
            pltpu.async_copy(val_a, win.at[idx_a], ssem_a, add=True)

            wait_load(2 * q + 1, idx_b, val_b, lsem_b)
            compute(idx_b, val_b)
            pltpu.async_copy(val_b, win.at[idx_b], ssem_b, add=True)

            @pl.when(q + 1 < NPAIR)
            def _():
                wait_scatter(val_a, ssem_a)
                load(2 * q + 2, idx_a, val_a, lsem_a)
                wait_scatter(val_b, ssem_b)
                load(2 * q + 3, idx_b, val_b, lsem_b)

            return carry1

        lax.fori_loop(0, NPAIR, pair_body, 0)
        wait_scatter(val_a, ssem_a)
        wait_scatter(val_b, ssem_b)
        plsc.subcore_barrier()

        @pl.when(wid != NWIN - 1)
        def _():
            pltpu.sync_copy(win.at[pl.ds(s * SLICE, SLICE)],
                            out_hbm.at[pl.ds(base + s * SLICE, SLICE)])

        @pl.when(wid == NWIN - 1)
        def _():
            lsl = LASTW // NT
            pltpu.sync_copy(win.at[pl.ds(s * lsl, lsl)],
                            out_hbm.at[pl.ds(base + s * lsl, lsl)])

        return carry

    lax.fori_loop(0, NPASS, pass_body, 0)


@jax.jit
def _unpool(idx, val):
    mesh = plsc.VectorSubcoreMesh(core_axis_name="c", subcore_axis_name="s")
    call = pl.kernel(
        _body,
        out_type=jax.ShapeDtypeStruct((TOTAL,), jnp.float32),
        mesh=mesh,
        scratch_types=[
            pltpu.VMEM_SHARED((W,), jnp.float32),
        ],
    )
    return call(idx, val)


def kernel(val, index):
    idx = jnp.reshape(index, (-1,)).astype(jnp.int32)
    v = jnp.reshape(val, (-1,))
    flat = _unpool(idx, v)
    return jnp.reshape(flat, (1, 512, 512, 96))


# speedup vs baseline: 11.6009x; 1.0320x over previous
import functools

import jax
import jax.numpy as jnp
from jax import lax
from jax.experimental import pallas as pl
from jax.experimental.pallas import tpu as pltpu
from jax.experimental.pallas import tpu_sc as plsc

N = 6291456
TOTAL = 25165824
W = 1835008
NWIN = 14
LASTW = TOTAL - 13 * W
NPASS = 7
NT = 16
PER_TILE = N // NT
CHUNK = 4096
NCHUNK = PER_TILE // CHUNK
NPAIR = NCHUNK // 2
ZB = CHUNK
SLICE = W // NT
DUMPMASK = (1 << 20) - 1


def _body(idx_hbm, val_hbm, out_hbm, win):
    pl.run_scoped(
        functools.partial(_scan, idx_hbm, val_hbm, out_hbm, win),
        pltpu.VMEM((CHUNK,), jnp.int32),
        pltpu.VMEM((CHUNK,), jnp.float32),
        pltpu.VMEM((CHUNK,), jnp.int32),
        pltpu.VMEM((CHUNK,), jnp.float32),
        pltpu.SemaphoreType.DMA,
        pltpu.SemaphoreType.DMA,
        pltpu.SemaphoreType.DMA,
        pltpu.SemaphoreType.DMA,
        pltpu.SemaphoreType.DMA,
    )


def _scan(idx_hbm, val_hbm, out_hbm, win, idx_a, val_a, idx_b, val_b,
          lsem_a, lsem_b, ssem_a, ssem_b, zsem):
    c = lax.axis_index("c")
    s = lax.axis_index("s")
    z16 = jnp.zeros((16,), jnp.float32)


    def load(k, ibuf, vbuf, sem):
        off = s * PER_TILE + k * CHUNK
        pltpu.async_copy(idx_hbm.at[pl.ds(off, CHUNK)], ibuf, sem)
        pltpu.async_copy(val_hbm.at[pl.ds(off, CHUNK)], vbuf, sem)

    def wait_load(k, ibuf, vbuf, sem):
        del k
        pltpu.make_async_copy(idx_hbm.at[pl.ds(0, CHUNK)], ibuf, sem).wait()
        pltpu.make_async_copy(val_hbm.at[pl.ds(0, CHUNK)], vbuf, sem).wait()

    def pass_body(p, carry):
        wid = 2 * p + c
        base = wid * W
        bound = jnp.where(wid == NWIN - 1, jnp.uint32(LASTW), jnp.uint32(W))

        def zb_init(i, carry):
            val_a[pl.ds(i * 16, 16)] = z16
            return carry

        lax.fori_loop(0, ZB // 16, zb_init, 0)
        for j in range(SLICE // ZB):
            pltpu.async_copy(val_a, win.at[pl.ds(s * SLICE + j * ZB, ZB)],
                             zsem)
        for j in range(SLICE // ZB):
            pltpu.make_async_copy(
                val_a, win.at[pl.ds(s * SLICE + j * ZB, ZB)], zsem).wait()
        plsc.subcore_barrier()

        load(0, idx_a, val_a, lsem_a)
        load(1, idx_b, val_b, lsem_b)

        def compute(ibuf, vbuf):
            def vec_body(i, carry2):
                for u in range(4):
                    o = (i * 4 + u) * 16
                    iv = ibuf[pl.ds(o, 16)]
                    vv = vbuf[pl.ds(o, 16)]
                    rel = iv - base
                    m = plsc.bitcast(rel, jnp.uint32) < bound
                    ibuf[pl.ds(o, 16)] = jnp.where(m, rel, rel & DUMPMASK)
                    vbuf[pl.ds(o, 16)] = jnp.where(m, vv, 0.0)
                return carry2

            lax.fori_loop(0, CHUNK // 64, vec_body, 0)

        def wait_scatter(vbuf, sem):
            pltpu.make_async_copy(vbuf, win.at[pl.ds(0, CHUNK)], sem).wait()

        def pair_body(q, carry1):
            wait_load(2 * q, idx_a, val_a, lsem_a)
            compute(idx_a, val_a)
            pltpu.async_copy(val_a, win.at[idx_a], ssem_a, add=True)

            wait_load(2 * q + 1, idx_b, val_b, lsem_b)
            compute(idx_b, val_b)
            pltpu.async_copy(val_b, win.at[idx_b], ssem_b, add=True)

            @pl.when(q + 1 < NPAIR)
            def _():
                wait_scatter(val_a, ssem_a)
                load(2 * q + 2, idx_a, val_a, lsem_a)
                wait_scatter(val_b, ssem_b)
                load(2 * q + 3, idx_b, val_b, lsem_b)

            return carry1

        lax.fori_loop(0, NPAIR, pair_body, 0)
        wait_scatter(val_a, ssem_a)
        wait_scatter(val_b, ssem_b)
        plsc.subcore_barrier()

        @pl.when(wid != NWIN - 1)
        def _():
            pltpu.sync_copy(win.at[pl.ds(s * SLICE, SLICE)],
                            out_hbm.at[pl.ds(base + s * SLICE, SLICE)])

        @pl.when(wid == NWIN - 1)
        def _():
            lsl = LASTW // NT
            pltpu.sync_copy(win.at[pl.ds(s * lsl, lsl)],
                            out_hbm.at[pl.ds(base + s * lsl, lsl)])

        return carry

    lax.fori_loop(0, NPASS, pass_body, 0)


@jax.jit
def _unpool(idx, val):
    mesh = plsc.VectorSubcoreMesh(core_axis_name="c", subcore_axis_name="s")
    call = pl.kernel(
        _body,
        out_type=jax.ShapeDtypeStruct((TOTAL,), jnp.float32),
        mesh=mesh,
        scratch_types=[
            pltpu.VMEM_SHARED((W,), jnp.float32),
        ],
    )
    return call(idx, val)


def kernel(val, index):
    idx = jnp.reshape(index, (-1,)).astype(jnp.int32)
    v = jnp.reshape(val, (-1,))
    flat = _unpool(idx, v)
    return jnp.reshape(flat, (1, 512, 512, 96))
